# Initial kernel scaffold; baseline (speedup 1.0000x reference)
import jax
import jax.numpy as jnp
from jax.experimental import pallas as pl


def kernel(x, pos, batch, feat_w1, feat_b1, feat_w2, feat_b2, glob_w1, glob_b1, glob_w2, glob_b2):
    raise NotImplementedError("write your pallas kernel here")


# speedup vs baseline: 19.6737x; 19.6737x over previous
import functools

import jax
import jax.numpy as jnp
from jax import lax
from jax.experimental import pallas as pl
from jax.experimental.pallas import tpu as pltpu
from jax.experimental.pallas import tpu_sc as plsc

R2 = 0.04
K = 64
NW = 32
CAP = 80


def _linear_kern(x_ref, w_ref, b_ref, o_ref):
    o_ref[...] = jnp.dot(x_ref[...], w_ref[...],
                         preferred_element_type=jnp.float32) + b_ref[...]


def _tc_linear(x, w, b):
    n, f = x.shape
    h = w.shape[1]
    blk = 1000
    return pl.pallas_call(
        _linear_kern,
        grid=(n // blk,),
        in_specs=[
            pl.BlockSpec((blk, f), lambda i: (i, 0)),
            pl.BlockSpec((f, h), lambda i: (0, 0)),
            pl.BlockSpec((1, h), lambda i: (0, 0)),
        ],
        out_specs=pl.BlockSpec((blk, h), lambda i: (i, 0)),
        out_shape=jax.ShapeDtypeStruct((n, h), jnp.float32),
    )(x, w, b)


def _merge_top64(t_d2, t_ix, c_d2, c_ix):
    out_d2, out_ix = [], []
    for t in range(4):
        rc_d2 = lax.rev(c_d2, (0,))
        rc_ix = lax.rev(c_ix, (0,))
        takes = t_d2[t] <= rc_d2
        lo_d2 = jnp.where(takes, t_d2[t], rc_d2)
        lo_ix = jnp.where(takes, t_ix[t], rc_ix)
        hi_d2 = jnp.where(takes, rc_d2, t_d2[t])
        hi_ix = jnp.where(takes, rc_ix, t_ix[t])
        nd, ni = plsc.sort_key_val(lo_d2, lo_ix)
        out_d2.append(nd)
        out_ix.append(ni)
        c_d2, c_ix = plsc.sort_key_val(hi_d2, hi_ix)
    return out_d2, out_ix


def _sload(ref, idx):
    return ref[pl.ds(idx, 16)][0]


def _round_bf16(v):
    u = lax.bitcast_convert_type(v, jnp.int32)
    r = (u + 0x7FFF + ((u >> 16) & 1)) & ~0xFFFF
    return lax.bitcast_convert_type(r, jnp.float32)


def _make_sc_kernel(n, npad, h, qpw):
    mesh = plsc.VectorSubcoreMesh(core_axis_name="c", subcore_axis_name="s")

    @functools.partial(
        pl.kernel,
        mesh=mesh,
        compiler_params=pltpu.CompilerParams(needs_layout_passes=False),
        out_type=[
            jax.ShapeDtypeStruct((n, K, h), jnp.float32),
            jax.ShapeDtypeStruct((npad * K,), jnp.float32),
            jax.ShapeDtypeStruct((npad * K,), jnp.float32),
            jax.ShapeDtypeStruct((npad * K,), jnp.float32),
        ],
        scratch_types=[
            pltpu.VMEM((npad,), jnp.float32),
            pltpu.VMEM((npad,), jnp.float32),
            pltpu.VMEM((npad,), jnp.float32),
            pltpu.VMEM((npad,), jnp.float32),
            pltpu.VMEM((qpw + 16,), jnp.int32),
            pltpu.VMEM((qpw + 16,), jnp.int32),
            pltpu.VMEM((CAP + 16,), jnp.int32),
            pltpu.VMEM((K,), jnp.int32),
            pltpu.VMEM((K, h), jnp.float32),
            pltpu.VMEM((qpw * K,), jnp.float32),
            pltpu.VMEM((qpw * K,), jnp.float32),
            pltpu.VMEM((qpw * K,), jnp.float32),
            pltpu.SemaphoreType.DMA,
        ],
    )
    def kern(px_hbm, py_hbm, pz_hbm, sq_hbm, qlo_hbm, qhi_hbm, g_hbm,
             gout_hbm, rx_hbm, ry_hbm, rz_hbm,
             px_v, py_v, pz_v, sq_v, qlo_v, qhi_v, buf_v, sel_v, grow_v,
             rxb_v, ryb_v, rzb_v, sem):
        lanes = lax.iota(jnp.int32, 16)
        wid = lax.axis_index("s") * 2 + lax.axis_index("c")
        base = wid * qpw
        pltpu.sync_copy(px_hbm, px_v)
        pltpu.sync_copy(py_hbm, py_v)
        pltpu.sync_copy(pz_hbm, pz_v)
        pltpu.sync_copy(sq_hbm, sq_v)
        pltpu.sync_copy(qlo_hbm.at[pl.ds(base, qpw)], qlo_v.at[pl.ds(0, qpw)])
        pltpu.sync_copy(qhi_hbm.at[pl.ds(base, qpw)], qhi_v.at[pl.ds(0, qpw)])
        nq = jnp.minimum(qpw, jnp.maximum(n - base, 0))

        def per_query(q, _):
            i = base + q
            lo = _sload(qlo_v, q)
            hi = _sload(qhi_v, q)
            qx = _sload(px_v, i)
            qy = _sload(py_v, i)
            qz = _sload(pz_v, i)
            qsq = _sload(sq_v, i)
            qxb = _round_bf16(px_v[pl.ds(i, 16)])[0]
            qyb = _round_bf16(py_v[pl.ds(i, 16)])[0]
            qzb = _round_bf16(pz_v[pl.ds(i, 16)])[0]
            k0 = (lo // 16) * 16
            nt = (hi - k0 + 15) // 16

            def d2_at(kvec):
                gxb = _round_bf16(plsc.load_gather(px_v, [kvec]))
                gyb = _round_bf16(plsc.load_gather(py_v, [kvec]))
                gzb = _round_bf16(plsc.load_gather(pz_v, [kvec]))
                sq = plsc.load_gather(sq_v, [kvec])
                dot = (qxb * gxb + qyb * gyb) + qzb * gzb
                return (qsq + sq) - 2.0 * dot

            def scan_body(t, cnt):
                kvec = k0 + t * 16 + lanes
                inb = (kvec >= lo) & (kvec < hi)
                kcl = jnp.where(inb, kvec, lo)
                d2 = d2_at(kcl)
                m = (d2 <= R2) & inb
                posn = cnt + jnp.cumsum(m.astype(jnp.int32)) - 1
                posn = jnp.minimum(posn, CAP + 15)
                plsc.store_scatter(buf_v, [posn], kcl, mask=m)
                return cnt + jnp.sum(m.astype(jnp.int32))

            cnt = lax.fori_loop(0, nt, scan_body, jnp.int32(0))

            def sparse_path(_):
                for t in range(4):
                    l = t * 16 + lanes
                    v = buf_v[pl.ds(t * 16, 16)]
                    sel_v[pl.ds(t * 16, 16)] = jnp.where(l < cnt, v, i)
                return 0

            def dense_path(_):
                t_d2 = [jnp.full((16,), jnp.inf, jnp.float32)] * 4
                t_ix = [jnp.full((16,), i, jnp.int32)] * 4

                def body(t, carry):
                    td0, td1, td2_, td3, ti0, ti1, ti2, ti3 = carry
                    kvec = k0 + t * 16 + lanes
                    inb = (kvec >= lo) & (kvec < hi)
                    kcl = jnp.where(inb, kvec, lo)
                    d2 = d2_at(kcl)
                    d2 = jnp.where((d2 <= R2) & inb, d2, jnp.inf)
                    thr = jnp.max(td3)

                    def do_merge(_):
                        cd, ci = plsc.sort_key_val(d2, kcl)
                        nd, ni = _merge_top64(
                            [td0, td1, td2_, td3], [ti0, ti1, ti2, ti3],
                            cd, ci)
                        return tuple(nd) + tuple(ni)

                    return lax.cond(jnp.min(d2) < thr, do_merge,
                                    lambda _: carry, 0)

                carry = lax.fori_loop(
                    0, nt, body, tuple(t_d2) + tuple(t_ix))
                for t in range(4):
                    sel_v[pl.ds(t * 16, 16)] = carry[4 + t]
                return 0

            lax.cond(cnt <= K, sparse_path, dense_path, 0)

            pltpu.async_copy(g_hbm.at[sel_v], grow_v, sem).wait()
            pltpu.sync_copy(grow_v, gout_hbm.at[i])

            for t in range(4):
                ids = sel_v[pl.ds(t * 16, 16)]
                off = q * K + t * 16
                rxb_v[pl.ds(off, 16)] = plsc.load_gather(px_v, [ids]) - qx
                ryb_v[pl.ds(off, 16)] = plsc.load_gather(py_v, [ids]) - qy
                rzb_v[pl.ds(off, 16)] = plsc.load_gather(pz_v, [ids]) - qz
            return 0

        lax.fori_loop(0, nq, per_query, 0)
        pltpu.sync_copy(rxb_v, rx_hbm.at[pl.ds(base * K, qpw * K)])
        pltpu.sync_copy(ryb_v, ry_hbm.at[pl.ds(base * K, qpw * K)])
        pltpu.sync_copy(rzb_v, rz_hbm.at[pl.ds(base * K, qpw * K)])

    return kern


def _msg_kern(g_ref, rx_ref, ry_ref, rz_ref, wp_ref, b2_ref,
              w2_ref, wg1_ref, bg1_ref, wg2_ref, bg2_ref, o_ref, *, qb, h):
    rel = jnp.concatenate(
        [rx_ref[...][:, :, None], ry_ref[...][:, :, None],
         rz_ref[...][:, :, None]], axis=-1).reshape(qb * K, 3)
    p = jnp.dot(rel, wp_ref[...], preferred_element_type=jnp.float32)
    a = jnp.maximum(g_ref[...].reshape(qb * K, h) + p, 0.0)
    m = jnp.dot(a, w2_ref[...], preferred_element_type=jnp.float32)
    agg = jnp.max(m.reshape(qb, K, h), axis=1) + b2_ref[...]
    o = jnp.maximum(
        jnp.dot(agg, wg1_ref[...], preferred_element_type=jnp.float32)
        + bg1_ref[...], 0.0)
    o_ref[...] = jnp.dot(o, wg2_ref[...],
                         preferred_element_type=jnp.float32) + bg2_ref[...]


def _tc_msg(G, RX, RY, RZ, wp, b2, w2, wg1, bg1, wg2, bg2):
    n, _, h = G.shape
    qb = 80
    grid = n // qb
    return pl.pallas_call(
        functools.partial(_msg_kern, qb=qb, h=h),
        grid=(grid,),
        in_specs=[
            pl.BlockSpec((qb, K, h), lambda i: (i, 0, 0)),
            pl.BlockSpec((qb, K), lambda i: (i, 0)),
            pl.BlockSpec((qb, K), lambda i: (i, 0)),
            pl.BlockSpec((qb, K), lambda i: (i, 0)),
            pl.BlockSpec((3, h), lambda i: (0, 0)),
            pl.BlockSpec((1, h), lambda i: (0, 0)),
            pl.BlockSpec((h, h), lambda i: (0, 0)),
            pl.BlockSpec((h, h), lambda i: (0, 0)),
            pl.BlockSpec((1, h), lambda i: (0, 0)),
            pl.BlockSpec((h, h), lambda i: (0, 0)),
            pl.BlockSpec((1, h), lambda i: (0, 0)),
        ],
        out_specs=pl.BlockSpec((qb, h), lambda i: (i, 0)),
        out_shape=jax.ShapeDtypeStruct((n, h), jnp.float32),
    )(G, RX, RY, RZ, wp, b2, w2, wg1, bg1, wg2, bg2)


def kernel(x, pos, batch, feat_w1, feat_b1, feat_w2, feat_b2,
           glob_w1, glob_b1, glob_w2, glob_b2):
    n, f = x.shape
    h = feat_w2.shape[0]
    qpw = ((n + NW - 1) // NW + 7) // 8 * 8
    npad = NW * qpw

    w1x = feat_w1[:f]
    wp = feat_w1[f:]
    g = _tc_linear(x, w1x, feat_b1.reshape(1, h))

    pad = npad - n
    px = jnp.pad(pos[:, 0], (0, pad))
    py = jnp.pad(pos[:, 1], (0, pad))
    pz = jnp.pad(pos[:, 2], (0, pad))
    sqv = jnp.pad(jnp.sum(pos * pos, axis=1), (0, pad))
    nb = 8
    seg = jnp.searchsorted(batch, jnp.arange(nb + 1, dtype=jnp.int32)
                           ).astype(jnp.int32)
    qlo = jnp.pad(seg[batch], (0, pad))
    qhi = jnp.pad(seg[batch + 1], (0, pad))

    G, RX, RY, RZ = _make_sc_kernel(n, npad, h, qpw)(
        px, py, pz, sqv, qlo, qhi, g)
    RX = RX.reshape(npad, K)
    RY = RY.reshape(npad, K)
    RZ = RZ.reshape(npad, K)

    out = _tc_msg(G, RX, RY, RZ, wp, feat_b2.reshape(1, h), feat_w2,
                  glob_w1, glob_b1.reshape(1, h), glob_w2,
                  glob_b2.reshape(1, h))
    return (out, pos, batch)


# speedup vs baseline: 20.1072x; 1.0220x over previous
import functools

import jax
import jax.numpy as jnp
from jax import lax
from jax.experimental import pallas as pl
from jax.experimental.pallas import tpu as pltpu
from jax.experimental.pallas import tpu_sc as plsc

R2 = 0.04
K = 64
NW = 32
CAP = 80


def _linear_kern(x_ref, w_ref, b_ref, o_ref):
    o_ref[...] = jnp.dot(x_ref[...], w_ref[...],
                         preferred_element_type=jnp.float32) + b_ref[...]


def _tc_linear(x, w, b):
    n, f = x.shape
    h = w.shape[1]
    blk = 1000
    return pl.pallas_call(
        _linear_kern,
        grid=(n // blk,),
        in_specs=[
            pl.BlockSpec((blk, f), lambda i: (i, 0)),
            pl.BlockSpec((f, h), lambda i: (0, 0)),
            pl.BlockSpec((1, h), lambda i: (0, 0)),
        ],
        out_specs=pl.BlockSpec((blk, h), lambda i: (i, 0)),
        out_shape=jax.ShapeDtypeStruct((n, h), jnp.float32),
    )(x, w, b)


def _merge_top64(t_d2, t_ix, c_d2, c_ix):
    out_d2, out_ix = [], []
    for t in range(4):
        rc_d2 = lax.rev(c_d2, (0,))
        rc_ix = lax.rev(c_ix, (0,))
        takes = t_d2[t] <= rc_d2
        lo_d2 = jnp.where(takes, t_d2[t], rc_d2)
        lo_ix = jnp.where(takes, t_ix[t], rc_ix)
        hi_d2 = jnp.where(takes, rc_d2, t_d2[t])
        hi_ix = jnp.where(takes, rc_ix, t_ix[t])
        nd, ni = plsc.sort_key_val(lo_d2, lo_ix)
        out_d2.append(nd)
        out_ix.append(ni)
        c_d2, c_ix = plsc.sort_key_val(hi_d2, hi_ix)
    return out_d2, out_ix


def _sload(ref, idx):
    return ref[pl.ds(idx, 16)][0]


def _round_bf16(v):
    u = lax.bitcast_convert_type(v, jnp.int32)
    r = (u + 0x7FFF + ((u >> 16) & 1)) & ~0xFFFF
    return lax.bitcast_convert_type(r, jnp.float32)


def _make_sc_kernel(n, npad, h, qpw):
    mesh = plsc.VectorSubcoreMesh(core_axis_name="c", subcore_axis_name="s")

    @functools.partial(
        pl.kernel,
        mesh=mesh,
        compiler_params=pltpu.CompilerParams(needs_layout_passes=False),
        out_type=[
            jax.ShapeDtypeStruct((n, K, h), jnp.float32),
            jax.ShapeDtypeStruct((npad * K,), jnp.float32),
            jax.ShapeDtypeStruct((npad * K,), jnp.float32),
            jax.ShapeDtypeStruct((npad * K,), jnp.float32),
        ],
        scratch_types=[
            pltpu.VMEM((npad,), jnp.float32),
            pltpu.VMEM((npad,), jnp.float32),
            pltpu.VMEM((npad,), jnp.float32),
            pltpu.VMEM((npad,), jnp.float32),
            pltpu.VMEM((qpw + 16,), jnp.int32),
            pltpu.VMEM((qpw + 16,), jnp.int32),
            pltpu.VMEM((CAP + 16,), jnp.int32),
            pltpu.VMEM((K,), jnp.int32),
            pltpu.VMEM((K, h), jnp.float32),
            pltpu.VMEM((qpw * K,), jnp.float32),
            pltpu.VMEM((qpw * K,), jnp.float32),
            pltpu.VMEM((qpw * K,), jnp.float32),
            pltpu.SemaphoreType.DMA,
        ],
    )
    def kern(px_hbm, py_hbm, pz_hbm, sq_hbm, qlo_hbm, qhi_hbm, g_hbm,
             gout_hbm, rx_hbm, ry_hbm, rz_hbm,
             px_v, py_v, pz_v, sq_v, qlo_v, qhi_v, buf_v, sel_v, grow_v,
             rxb_v, ryb_v, rzb_v, sem):
        lanes = lax.iota(jnp.int32, 16)
        wid = lax.axis_index("s") * 2 + lax.axis_index("c")
        base = wid * qpw
        pltpu.sync_copy(px_hbm, px_v)
        pltpu.sync_copy(py_hbm, py_v)
        pltpu.sync_copy(pz_hbm, pz_v)
        pltpu.sync_copy(sq_hbm, sq_v)
        pltpu.sync_copy(qlo_hbm.at[pl.ds(base, qpw)], qlo_v.at[pl.ds(0, qpw)])
        pltpu.sync_copy(qhi_hbm.at[pl.ds(base, qpw)], qhi_v.at[pl.ds(0, qpw)])
        nq = jnp.minimum(qpw, jnp.maximum(n - base, 0))

        def per_query(q, _):
            i = base + q
            lo = _sload(qlo_v, q)
            hi = _sload(qhi_v, q)
            qx = _sload(px_v, i)
            qy = _sload(py_v, i)
            qz = _sload(pz_v, i)
            qsq = _sload(sq_v, i)
            qxb = _round_bf16(px_v[pl.ds(i, 16)])[0]
            qyb = _round_bf16(py_v[pl.ds(i, 16)])[0]
            qzb = _round_bf16(pz_v[pl.ds(i, 16)])[0]
            k0 = (lo // 16) * 16
            nt = (hi - k0 + 15) // 16

            def d2_at(koff):
                gxb = _round_bf16(px_v[pl.ds(koff, 16)])
                gyb = _round_bf16(py_v[pl.ds(koff, 16)])
                gzb = _round_bf16(pz_v[pl.ds(koff, 16)])
                sq = sq_v[pl.ds(koff, 16)]
                dot = (qxb * gxb + qyb * gyb) + qzb * gzb
                return (qsq + sq) - 2.0 * dot

            def scan_body(t, cnt):
                koff = k0 + t * 16
                kvec = koff + lanes
                inb = (kvec >= lo) & (kvec < hi)
                d2 = d2_at(koff)
                m = (d2 <= R2) & inb
                csum = jnp.cumsum(m.astype(jnp.int32))
                posn = jnp.minimum(cnt + csum - 1, CAP + 15)
                plsc.store_scatter(buf_v, [posn], kvec, mask=m)
                return cnt + csum[15]

            cnt = lax.fori_loop(0, nt, scan_body, jnp.int32(0))

            def sparse_path(_):
                for t in range(4):
                    l = t * 16 + lanes
                    v = buf_v[pl.ds(t * 16, 16)]
                    sel_v[pl.ds(t * 16, 16)] = jnp.where(l < cnt, v, i)
                return 0

            def dense_path(_):
                t_d2 = [jnp.full((16,), jnp.inf, jnp.float32)] * 4
                t_ix = [jnp.full((16,), i, jnp.int32)] * 4

                def body(t, carry):
                    td0, td1, td2_, td3, ti0, ti1, ti2, ti3 = carry
                    koff = k0 + t * 16
                    kvec = koff + lanes
                    inb = (kvec >= lo) & (kvec < hi)
                    d2 = d2_at(koff)
                    d2 = jnp.where((d2 <= R2) & inb, d2, jnp.inf)
                    thr = jnp.max(td3)

                    def do_merge(_):
                        cd, ci = plsc.sort_key_val(d2, kvec)
                        nd, ni = _merge_top64(
                            [td0, td1, td2_, td3], [ti0, ti1, ti2, ti3],
                            cd, ci)
                        return tuple(nd) + tuple(ni)

                    return lax.cond(jnp.min(d2) < thr, do_merge,
                                    lambda _: carry, 0)

                carry = lax.fori_loop(
                    0, nt, body, tuple(t_d2) + tuple(t_ix))
                for t in range(4):
                    sel_v[pl.ds(t * 16, 16)] = carry[4 + t]
                return 0

            lax.cond(cnt <= K, sparse_path, dense_path, 0)

            pltpu.async_copy(g_hbm.at[sel_v], grow_v, sem).wait()
            pltpu.sync_copy(grow_v, gout_hbm.at[i])

            for t in range(4):
                ids = sel_v[pl.ds(t * 16, 16)]
                off = q * K + t * 16
                rxb_v[pl.ds(off, 16)] = plsc.load_gather(px_v, [ids]) - qx
                ryb_v[pl.ds(off, 16)] = plsc.load_gather(py_v, [ids]) - qy
                rzb_v[pl.ds(off, 16)] = plsc.load_gather(pz_v, [ids]) - qz
            return 0

        lax.fori_loop(0, nq, per_query, 0)
        pltpu.sync_copy(rxb_v, rx_hbm.at[pl.ds(base * K, qpw * K)])
        pltpu.sync_copy(ryb_v, ry_hbm.at[pl.ds(base * K, qpw * K)])
        pltpu.sync_copy(rzb_v, rz_hbm.at[pl.ds(base * K, qpw * K)])

    return kern


def _msg_kern(g_ref, rx_ref, ry_ref, rz_ref, wp_ref, b2_ref,
              w2_ref, wg1_ref, bg1_ref, wg2_ref, bg2_ref, o_ref, *, qb, h):
    rel = jnp.concatenate(
        [rx_ref[...][:, :, None], ry_ref[...][:, :, None],
         rz_ref[...][:, :, None]], axis=-1).reshape(qb * K, 3)
    p = jnp.dot(rel, wp_ref[...], preferred_element_type=jnp.float32)
    a = jnp.maximum(g_ref[...].reshape(qb * K, h) + p, 0.0)
    m = jnp.dot(a, w2_ref[...], preferred_element_type=jnp.float32)
    agg = jnp.max(m.reshape(qb, K, h), axis=1) + b2_ref[...]
    o = jnp.maximum(
        jnp.dot(agg, wg1_ref[...], preferred_element_type=jnp.float32)
        + bg1_ref[...], 0.0)
    o_ref[...] = jnp.dot(o, wg2_ref[...],
                         preferred_element_type=jnp.float32) + bg2_ref[...]


def _tc_msg(G, RX, RY, RZ, wp, b2, w2, wg1, bg1, wg2, bg2):
    n, _, h = G.shape
    qb = 80
    grid = n // qb
    return pl.pallas_call(
        functools.partial(_msg_kern, qb=qb, h=h),
        grid=(grid,),
        in_specs=[
            pl.BlockSpec((qb, K, h), lambda i: (i, 0, 0)),
            pl.BlockSpec((qb, K), lambda i: (i, 0)),
            pl.BlockSpec((qb, K), lambda i: (i, 0)),
            pl.BlockSpec((qb, K), lambda i: (i, 0)),
            pl.BlockSpec((3, h), lambda i: (0, 0)),
            pl.BlockSpec((1, h), lambda i: (0, 0)),
            pl.BlockSpec((h, h), lambda i: (0, 0)),
            pl.BlockSpec((h, h), lambda i: (0, 0)),
            pl.BlockSpec((1, h), lambda i: (0, 0)),
            pl.BlockSpec((h, h), lambda i: (0, 0)),
            pl.BlockSpec((1, h), lambda i: (0, 0)),
        ],
        out_specs=pl.BlockSpec((qb, h), lambda i: (i, 0)),
        out_shape=jax.ShapeDtypeStruct((n, h), jnp.float32),
    )(G, RX, RY, RZ, wp, b2, w2, wg1, bg1, wg2, bg2)


def kernel(x, pos, batch, feat_w1, feat_b1, feat_w2, feat_b2,
           glob_w1, glob_b1, glob_w2, glob_b2):
    n, f = x.shape
    h = feat_w2.shape[0]
    qpw = ((n + NW - 1) // NW + 7) // 8 * 8
    npad = NW * qpw

    w1x = feat_w1[:f]
    wp = feat_w1[f:]
    g = _tc_linear(x, w1x, feat_b1.reshape(1, h))

    pad = npad - n
    px = jnp.pad(pos[:, 0], (0, pad))
    py = jnp.pad(pos[:, 1], (0, pad))
    pz = jnp.pad(pos[:, 2], (0, pad))
    sqv = jnp.pad(jnp.sum(pos * pos, axis=1), (0, pad))
    nb = 8
    seg = jnp.searchsorted(batch, jnp.arange(nb + 1, dtype=jnp.int32)
                           ).astype(jnp.int32)
    qlo = jnp.pad(seg[batch], (0, pad))
    qhi = jnp.pad(seg[batch + 1], (0, pad))

    G, RX, RY, RZ = _make_sc_kernel(n, npad, h, qpw)(
        px, py, pz, sqv, qlo, qhi, g)
    RX = RX.reshape(npad, K)
    RY = RY.reshape(npad, K)
    RZ = RZ.reshape(npad, K)

    out = _tc_msg(G, RX, RY, RZ, wp, feat_b2.reshape(1, h), feat_w2,
                  glob_w1, glob_b1.reshape(1, h), glob_w2,
                  glob_b2.reshape(1, h))
    return (out, pos, batch)


# speedup vs baseline: 26.4148x; 1.3137x over previous
import functools

import jax
import jax.numpy as jnp
from jax import lax
from jax.experimental import pallas as pl
from jax.experimental.pallas import tpu as pltpu
from jax.experimental.pallas import tpu_sc as plsc

R2 = 0.04
K = 64
NW = 32
CAP = 80


def _linear_kern(x_ref, w_ref, b_ref, o_ref):
    o_ref[...] = jnp.dot(x_ref[...], w_ref[...],
                         preferred_element_type=jnp.float32) + b_ref[...]


def _tc_linear(x, w, b):
    n, f = x.shape
    h = w.shape[1]
    blk = 1000
    return pl.pallas_call(
        _linear_kern,
        grid=(n // blk,),
        in_specs=[
            pl.BlockSpec((blk, f), lambda i: (i, 0)),
            pl.BlockSpec((f, h), lambda i: (0, 0)),
            pl.BlockSpec((1, h), lambda i: (0, 0)),
        ],
        out_specs=pl.BlockSpec((blk, h), lambda i: (i, 0)),
        out_shape=jax.ShapeDtypeStruct((n, h), jnp.float32),
    )(x, w, b)


def _merge_top64(t_d2, t_ix, c_d2, c_ix):
    out_d2, out_ix = [], []
    for t in range(4):
        rc_d2 = lax.rev(c_d2, (0,))
        rc_ix = lax.rev(c_ix, (0,))
        takes = t_d2[t] <= rc_d2
        lo_d2 = jnp.where(takes, t_d2[t], rc_d2)
        lo_ix = jnp.where(takes, t_ix[t], rc_ix)
        hi_d2 = jnp.where(takes, rc_d2, t_d2[t])
        hi_ix = jnp.where(takes, rc_ix, t_ix[t])
        nd, ni = plsc.sort_key_val(lo_d2, lo_ix)
        out_d2.append(nd)
        out_ix.append(ni)
        c_d2, c_ix = plsc.sort_key_val(hi_d2, hi_ix)
    return out_d2, out_ix


def _sload(ref, idx):
    return ref[pl.ds(idx, 16)][0]


def _round_bf16(v):
    u = lax.bitcast_convert_type(v, jnp.int32)
    r = (u + 0x7FFF + ((u >> 16) & 1)) & ~0xFFFF
    return lax.bitcast_convert_type(r, jnp.float32)


def _make_sc_kernel(n, npad, h, qpw):
    mesh = plsc.VectorSubcoreMesh(core_axis_name="c", subcore_axis_name="s")

    @functools.partial(
        pl.kernel,
        mesh=mesh,
        compiler_params=pltpu.CompilerParams(needs_layout_passes=False),
        out_type=[
            jax.ShapeDtypeStruct((n, K, h), jnp.float32),
            jax.ShapeDtypeStruct((npad * K,), jnp.float32),
            jax.ShapeDtypeStruct((npad * K,), jnp.float32),
            jax.ShapeDtypeStruct((npad * K,), jnp.float32),
        ],
        scratch_types=[
            pltpu.VMEM((npad,), jnp.float32),
            pltpu.VMEM((npad,), jnp.float32),
            pltpu.VMEM((npad,), jnp.float32),
            pltpu.VMEM((npad,), jnp.float32),
            pltpu.VMEM((qpw + 16,), jnp.int32),
            pltpu.VMEM((qpw + 16,), jnp.int32),
            pltpu.VMEM((CAP + 16,), jnp.int32),
            pltpu.VMEM((K,), jnp.int32),
            pltpu.VMEM((K,), jnp.int32),
            pltpu.VMEM((K, h), jnp.float32),
            pltpu.VMEM((K, h), jnp.float32),
            pltpu.VMEM((qpw * K,), jnp.float32),
            pltpu.VMEM((qpw * K,), jnp.float32),
            pltpu.VMEM((qpw * K,), jnp.float32),
            pltpu.SemaphoreType.DMA,
            pltpu.SemaphoreType.DMA,
            pltpu.SemaphoreType.DMA,
        ],
    )
    def kern(px_hbm, py_hbm, pz_hbm, sq_hbm, qlo_hbm, qhi_hbm, g_hbm,
             gout_hbm, rx_hbm, ry_hbm, rz_hbm,
             px_v, py_v, pz_v, sq_v, qlo_v, qhi_v, buf_v, selA_v, selB_v,
             growA_v, growB_v, rxb_v, ryb_v, rzb_v, gsemA, gsemB, ssem):
        lanes = lax.iota(jnp.int32, 16)
        wid = lax.axis_index("s") * 2 + lax.axis_index("c")
        base = wid * qpw
        pltpu.sync_copy(px_hbm, px_v)
        pltpu.sync_copy(py_hbm, py_v)
        pltpu.sync_copy(pz_hbm, pz_v)
        pltpu.sync_copy(sq_hbm, sq_v)
        pltpu.sync_copy(qlo_hbm.at[pl.ds(base, qpw)], qlo_v.at[pl.ds(0, qpw)])
        pltpu.sync_copy(qhi_hbm.at[pl.ds(base, qpw)], qhi_v.at[pl.ds(0, qpw)])
        nq = jnp.minimum(qpw, jnp.maximum(n - base, 0))

        def select_into(q, sel_v):
            i = base + q
            lo = _sload(qlo_v, q)
            hi = _sload(qhi_v, q)
            qx = _sload(px_v, i)
            qy = _sload(py_v, i)
            qz = _sload(pz_v, i)
            qsq = _sload(sq_v, i)
            qxb = _round_bf16(px_v[pl.ds(i, 16)])[0]
            qyb = _round_bf16(py_v[pl.ds(i, 16)])[0]
            qzb = _round_bf16(pz_v[pl.ds(i, 16)])[0]
            k0 = (lo // 16) * 16
            nt = (hi - k0 + 15) // 16

            def d2_at(koff):
                gxb = _round_bf16(px_v[pl.ds(koff, 16)])
                gyb = _round_bf16(py_v[pl.ds(koff, 16)])
                gzb = _round_bf16(pz_v[pl.ds(koff, 16)])
                sq = sq_v[pl.ds(koff, 16)]
                dot = (qxb * gxb + qyb * gyb) + qzb * gzb
                return (qsq + sq) - 2.0 * dot

            def scan_body(t, cnt):
                koff = k0 + t * 16
                kvec = koff + lanes
                inb = (kvec >= lo) & (kvec < hi)
                d2 = d2_at(koff)
                m = (d2 <= R2) & inb
                csum = jnp.cumsum(m.astype(jnp.int32))
                posn = jnp.minimum(cnt + csum - 1, CAP + 15)
                plsc.store_scatter(buf_v, [posn], kvec, mask=m)
                return cnt + csum[15]

            cnt = lax.fori_loop(0, nt, scan_body, jnp.int32(0))

            def sparse_path(_):
                for t in range(4):
                    l = t * 16 + lanes
                    v = buf_v[pl.ds(t * 16, 16)]
                    sel_v[pl.ds(t * 16, 16)] = jnp.where(l < cnt, v, i)
                return 0

            def dense_path(_):
                t_d2 = [jnp.full((16,), jnp.inf, jnp.float32)] * 4
                t_ix = [jnp.full((16,), i, jnp.int32)] * 4

                def body(t, carry):
                    td0, td1, td2_, td3, ti0, ti1, ti2, ti3 = carry
                    koff = k0 + t * 16
                    kvec = koff + lanes
                    inb = (kvec >= lo) & (kvec < hi)
                    d2 = d2_at(koff)
                    d2 = jnp.where((d2 <= R2) & inb, d2, jnp.inf)
                    thr = jnp.max(td3)

                    def do_merge(_):
                        cd, ci = plsc.sort_key_val(d2, kvec)
                        nd, ni = _merge_top64(
                            [td0, td1, td2_, td3], [ti0, ti1, ti2, ti3],
                            cd, ci)
                        return tuple(nd) + tuple(ni)

                    return lax.cond(jnp.min(d2) < thr, do_merge,
                                    lambda _: carry, 0)

                carry = lax.fori_loop(
                    0, nt, body, tuple(t_d2) + tuple(t_ix))
                for t in range(4):
                    sel_v[pl.ds(t * 16, 16)] = carry[4 + t]
                return 0

            lax.cond(cnt <= K, sparse_path, dense_path, 0)

            for t in range(4):
                ids = sel_v[pl.ds(t * 16, 16)]
                off = q * K + t * 16
                rxb_v[pl.ds(off, 16)] = plsc.load_gather(px_v, [ids]) - qx
                ryb_v[pl.ds(off, 16)] = plsc.load_gather(py_v, [ids]) - qy
                rzb_v[pl.ds(off, 16)] = plsc.load_gather(pz_v, [ids]) - qz

        def drain_two_stores():
            pltpu.make_async_copy(growA_v, gout_hbm.at[0], ssem).wait()
            pltpu.make_async_copy(growB_v, gout_hbm.at[0], ssem).wait()

        def per_pair(p, _):
            q0 = 2 * p
            q1 = q0 + 1
            select_into(q0, selA_v)

            @pl.when(p > 0)
            def _():
                drain_two_stores()

            cpA = pltpu.async_copy(g_hbm.at[selA_v], growA_v, gsemA)
            select_into(q1, selB_v)
            cpB = pltpu.async_copy(g_hbm.at[selB_v], growB_v, gsemB)
            cpA.wait()
            pltpu.async_copy(growA_v, gout_hbm.at[base + q0], ssem)
            cpB.wait()
            pltpu.async_copy(growB_v, gout_hbm.at[base + q1], ssem)
            return 0

        lax.fori_loop(0, nq // 2, per_pair, 0)
        drain_two_stores()
        pltpu.sync_copy(rxb_v, rx_hbm.at[pl.ds(base * K, qpw * K)])
        pltpu.sync_copy(ryb_v, ry_hbm.at[pl.ds(base * K, qpw * K)])
        pltpu.sync_copy(rzb_v, rz_hbm.at[pl.ds(base * K, qpw * K)])

    return kern


def _msg_kern(g_ref, rx_ref, ry_ref, rz_ref, wp_ref, b2_ref,
              w2_ref, wg1_ref, bg1_ref, wg2_ref, bg2_ref, o_ref, *, qb, h):
    rel = jnp.concatenate(
        [rx_ref[...][:, :, None], ry_ref[...][:, :, None],
         rz_ref[...][:, :, None]], axis=-1).reshape(qb * K, 3)
    p = jnp.dot(rel, wp_ref[...], preferred_element_type=jnp.float32)
    a = jnp.maximum(g_ref[...].reshape(qb * K, h) + p, 0.0)
    m = jnp.dot(a, w2_ref[...], preferred_element_type=jnp.float32)
    agg = jnp.max(m.reshape(qb, K, h), axis=1) + b2_ref[...]
    o = jnp.maximum(
        jnp.dot(agg, wg1_ref[...], preferred_element_type=jnp.float32)
        + bg1_ref[...], 0.0)
    o_ref[...] = jnp.dot(o, wg2_ref[...],
                         preferred_element_type=jnp.float32) + bg2_ref[...]


def _tc_msg(G, RX, RY, RZ, wp, b2, w2, wg1, bg1, wg2, bg2):
    n, _, h = G.shape
    qb = 80
    grid = n // qb
    return pl.pallas_call(
        functools.partial(_msg_kern, qb=qb, h=h),
        grid=(grid,),
        in_specs=[
            pl.BlockSpec((qb, K, h), lambda i: (i, 0, 0)),
            pl.BlockSpec((qb, K), lambda i: (i, 0)),
            pl.BlockSpec((qb, K), lambda i: (i, 0)),
            pl.BlockSpec((qb, K), lambda i: (i, 0)),
            pl.BlockSpec((3, h), lambda i: (0, 0)),
            pl.BlockSpec((1, h), lambda i: (0, 0)),
            pl.BlockSpec((h, h), lambda i: (0, 0)),
            pl.BlockSpec((h, h), lambda i: (0, 0)),
            pl.BlockSpec((1, h), lambda i: (0, 0)),
            pl.BlockSpec((h, h), lambda i: (0, 0)),
            pl.BlockSpec((1, h), lambda i: (0, 0)),
        ],
        out_specs=pl.BlockSpec((qb, h), lambda i: (i, 0)),
        out_shape=jax.ShapeDtypeStruct((n, h), jnp.float32),
    )(G, RX, RY, RZ, wp, b2, w2, wg1, bg1, wg2, bg2)


def kernel(x, pos, batch, feat_w1, feat_b1, feat_w2, feat_b2,
           glob_w1, glob_b1, glob_w2, glob_b2):
    n, f = x.shape
    h = feat_w2.shape[0]
    qpw = ((n + NW - 1) // NW + 7) // 8 * 8
    npad = NW * qpw

    w1x = feat_w1[:f]
    wp = feat_w1[f:]
    g = _tc_linear(x, w1x, feat_b1.reshape(1, h))

    pad = npad - n
    px = jnp.pad(pos[:, 0], (0, pad))
    py = jnp.pad(pos[:, 1], (0, pad))
    pz = jnp.pad(pos[:, 2], (0, pad))
    sqv = jnp.pad(jnp.sum(pos * pos, axis=1), (0, pad))
    nb = 8
    seg = jnp.searchsorted(batch, jnp.arange(nb + 1, dtype=jnp.int32)
                           ).astype(jnp.int32)
    qlo = jnp.pad(seg[batch], (0, pad))
    qhi = jnp.pad(seg[batch + 1], (0, pad))

    G, RX, RY, RZ = _make_sc_kernel(n, npad, h, qpw)(
        px, py, pz, sqv, qlo, qhi, g)
    RX = RX.reshape(npad, K)
    RY = RY.reshape(npad, K)
    RZ = RZ.reshape(npad, K)

    out = _tc_msg(G, RX, RY, RZ, wp, feat_b2.reshape(1, h), feat_w2,
                  glob_w1, glob_b1.reshape(1, h), glob_w2,
                  glob_b2.reshape(1, h))
    return (out, pos, batch)


# speedup vs baseline: 29.6674x; 1.1231x over previous
import functools

import jax
import jax.numpy as jnp
from jax import lax
from jax.experimental import pallas as pl
from jax.experimental.pallas import tpu as pltpu
from jax.experimental.pallas import tpu_sc as plsc

R2 = 0.04
K = 64
NW = 32
CAP = 80
RCH = 32


def _linear_kern(x_ref, w_ref, b_ref, o_ref):
    o_ref[...] = jnp.dot(x_ref[...], w_ref[...],
                         preferred_element_type=jnp.float32) + b_ref[...]


def _tc_linear(x, w, b):
    n, f = x.shape
    h = w.shape[1]
    blk = 1000
    return pl.pallas_call(
        _linear_kern,
        grid=(n // blk,),
        in_specs=[
            pl.BlockSpec((blk, f), lambda i: (i, 0)),
            pl.BlockSpec((f, h), lambda i: (0, 0)),
            pl.BlockSpec((1, h), lambda i: (0, 0)),
        ],
        out_specs=pl.BlockSpec((blk, h), lambda i: (i, 0)),
        out_shape=jax.ShapeDtypeStruct((n, h), jnp.float32),
    )(x, w, b)


def _merge_top64(t_d2, t_ix, c_d2, c_ix):
    out_d2, out_ix = [], []
    for t in range(4):
        rc_d2 = lax.rev(c_d2, (0,))
        rc_ix = lax.rev(c_ix, (0,))
        takes = t_d2[t] <= rc_d2
        lo_d2 = jnp.where(takes, t_d2[t], rc_d2)
        lo_ix = jnp.where(takes, t_ix[t], rc_ix)
        hi_d2 = jnp.where(takes, rc_d2, t_d2[t])
        hi_ix = jnp.where(takes, rc_ix, t_ix[t])
        nd, ni = plsc.sort_key_val(lo_d2, lo_ix)
        out_d2.append(nd)
        out_ix.append(ni)
        c_d2, c_ix = plsc.sort_key_val(hi_d2, hi_ix)
    return out_d2, out_ix


def _sload(ref, idx):
    return ref[pl.ds(idx, 16)][0]


def _make_sc_kernel(n, npad, h, qpw):
    mesh = plsc.VectorSubcoreMesh(core_axis_name="c", subcore_axis_name="s")

    @functools.partial(
        pl.kernel,
        mesh=mesh,
        compiler_params=pltpu.CompilerParams(needs_layout_passes=False),
        out_type=[
            jax.ShapeDtypeStruct((n, K, h), jnp.float32),
            jax.ShapeDtypeStruct((npad * K,), jnp.float32),
            jax.ShapeDtypeStruct((npad * K,), jnp.float32),
            jax.ShapeDtypeStruct((npad * K,), jnp.float32),
        ],
        scratch_types=[
            pltpu.VMEM((npad,), jnp.float32),
            pltpu.VMEM((npad,), jnp.float32),
            pltpu.VMEM((npad,), jnp.float32),
            pltpu.VMEM((npad,), jnp.float32),
            pltpu.VMEM((npad,), jnp.float32),
            pltpu.VMEM((npad,), jnp.float32),
            pltpu.VMEM((npad,), jnp.float32),
            pltpu.VMEM((qpw + 16,), jnp.int32),
            pltpu.VMEM((qpw + 16,), jnp.int32),
            pltpu.VMEM((CAP + 32,), jnp.int32),
            pltpu.VMEM((K,), jnp.int32),
            pltpu.VMEM((K,), jnp.int32),
            pltpu.VMEM((K, h), jnp.float32),
            pltpu.VMEM((K, h), jnp.float32),
            pltpu.VMEM((RCH * K,), jnp.float32),
            pltpu.VMEM((RCH * K,), jnp.float32),
            pltpu.VMEM((RCH * K,), jnp.float32),
            pltpu.SemaphoreType.DMA,
            pltpu.SemaphoreType.DMA,
            pltpu.SemaphoreType.DMA,
        ],
    )
    def kern(px_hbm, py_hbm, pz_hbm, pxb_hbm, pyb_hbm, pzb_hbm, sq_hbm,
             qlo_hbm, qhi_hbm, g_hbm,
             gout_hbm, rx_hbm, ry_hbm, rz_hbm,
             px_v, py_v, pz_v, pxb_v, pyb_v, pzb_v, sq_v, qlo_v, qhi_v,
             buf_v, selA_v, selB_v,
             growA_v, growB_v, rxb_v, ryb_v, rzb_v, gsemA, gsemB, ssem):
        lanes = lax.iota(jnp.int32, 16)
        wid = lax.axis_index("s") * 2 + lax.axis_index("c")
        base = wid * qpw
        pltpu.sync_copy(px_hbm, px_v)
        pltpu.sync_copy(py_hbm, py_v)
        pltpu.sync_copy(pz_hbm, pz_v)
        pltpu.sync_copy(pxb_hbm, pxb_v)
        pltpu.sync_copy(pyb_hbm, pyb_v)
        pltpu.sync_copy(pzb_hbm, pzb_v)
        pltpu.sync_copy(sq_hbm, sq_v)
        pltpu.sync_copy(qlo_hbm.at[pl.ds(base, qpw)], qlo_v.at[pl.ds(0, qpw)])
        pltpu.sync_copy(qhi_hbm.at[pl.ds(base, qpw)], qhi_v.at[pl.ds(0, qpw)])
        nq = jnp.minimum(qpw, jnp.maximum(n - base, 0))

        def select_into(q, sel_v):
            i = base + q
            lo = _sload(qlo_v, q)
            hi = _sload(qhi_v, q)
            qx = _sload(px_v, i)
            qy = _sload(py_v, i)
            qz = _sload(pz_v, i)
            qsq = _sload(sq_v, i)
            qxb = _sload(pxb_v, i)
            qyb = _sload(pyb_v, i)
            qzb = _sload(pzb_v, i)
            k0 = (lo // 16) * 16
            nt = (hi - k0 + 15) // 16

            def d2_at(koff):
                gxb = pxb_v[pl.ds(koff, 16)]
                gyb = pyb_v[pl.ds(koff, 16)]
                gzb = pzb_v[pl.ds(koff, 16)]
                sq = sq_v[pl.ds(koff, 16)]
                dot = (qxb * gxb + qyb * gyb) + qzb * gzb
                return (qsq + sq) - 2.0 * dot

            def scan_body(t, cnt):
                koff = k0 + t * 16
                kvec = koff + lanes
                inb = (kvec >= lo) & (kvec < hi)
                d2 = d2_at(koff)
                m = (d2 <= R2) & inb
                cs = jnp.minimum(cnt, CAP)
                plsc.store_compressed(buf_v.at[pl.ds(cs, 16)], kvec, mask=m)
                return cnt + plsc.all_reduce_population_count(m)[0]

            cnt = lax.fori_loop(0, nt, scan_body, jnp.int32(0))

            def sparse_path(_):
                for t in range(4):
                    l = t * 16 + lanes
                    v = buf_v[pl.ds(t * 16, 16)]
                    sel_v[pl.ds(t * 16, 16)] = jnp.where(l < cnt, v, i)
                return 0

            def dense_path(_):
                t_d2 = [jnp.full((16,), jnp.inf, jnp.float32)] * 4
                t_ix = [jnp.full((16,), i, jnp.int32)] * 4

                def body(t, carry):
                    td0, td1, td2_, td3, ti0, ti1, ti2, ti3 = carry
                    koff = k0 + t * 16
                    kvec = koff + lanes
                    inb = (kvec >= lo) & (kvec < hi)
                    d2 = d2_at(koff)
                    d2 = jnp.where((d2 <= R2) & inb, d2, jnp.inf)
                    thr = jnp.max(td3)

                    def do_merge(_):
                        cd, ci = plsc.sort_key_val(d2, kvec)
                        nd, ni = _merge_top64(
                            [td0, td1, td2_, td3], [ti0, ti1, ti2, ti3],
                            cd, ci)
                        return tuple(nd) + tuple(ni)

                    return lax.cond(jnp.min(d2) < thr, do_merge,
                                    lambda _: carry, 0)

                carry = lax.fori_loop(
                    0, nt, body, tuple(t_d2) + tuple(t_ix))
                for t in range(4):
                    sel_v[pl.ds(t * 16, 16)] = carry[4 + t]
                return 0

            lax.cond(cnt <= K, sparse_path, dense_path, 0)

            qo = (q % RCH) * K
            for t in range(4):
                ids = sel_v[pl.ds(t * 16, 16)]
                off = qo + t * 16
                rxb_v[pl.ds(off, 16)] = plsc.load_gather(px_v, [ids]) - qx
                ryb_v[pl.ds(off, 16)] = plsc.load_gather(py_v, [ids]) - qy
                rzb_v[pl.ds(off, 16)] = plsc.load_gather(pz_v, [ids]) - qz

        def flush_rel(c):
            off = (base + c * RCH) * K
            pltpu.sync_copy(rxb_v, rx_hbm.at[pl.ds(off, RCH * K)])
            pltpu.sync_copy(ryb_v, ry_hbm.at[pl.ds(off, RCH * K)])
            pltpu.sync_copy(rzb_v, rz_hbm.at[pl.ds(off, RCH * K)])

        def drain_two_stores():
            pltpu.make_async_copy(growA_v, gout_hbm.at[0], ssem).wait()
            pltpu.make_async_copy(growB_v, gout_hbm.at[0], ssem).wait()

        def per_pair(p, _):
            q0 = 2 * p
            q1 = q0 + 1
            select_into(q0, selA_v)

            @pl.when(p > 0)
            def _():
                drain_two_stores()

            cpA = pltpu.async_copy(g_hbm.at[selA_v], growA_v, gsemA)
            select_into(q1, selB_v)
            cpB = pltpu.async_copy(g_hbm.at[selB_v], growB_v, gsemB)
            cpA.wait()
            pltpu.async_copy(growA_v, gout_hbm.at[base + q0], ssem)
            cpB.wait()
            pltpu.async_copy(growB_v, gout_hbm.at[base + q1], ssem)

            @pl.when((p % (RCH // 2)) == RCH // 2 - 1)
            def _():
                flush_rel(p // (RCH // 2))

            return 0

        lax.fori_loop(0, nq // 2, per_pair, 0)
        drain_two_stores()
        flush_rel((nq // 2 - 1) // (RCH // 2))

    return kern


def _msg_kern(g_ref, rx_ref, ry_ref, rz_ref, wp_ref, b2_ref,
              w2_ref, wg1_ref, bg1_ref, wg2_ref, bg2_ref, o_ref, *, qb, h):
    rel = jnp.concatenate(
        [rx_ref[...][:, :, None], ry_ref[...][:, :, None],
         rz_ref[...][:, :, None]], axis=-1).reshape(qb * K, 3)
    p = jnp.dot(rel, wp_ref[...], preferred_element_type=jnp.float32)
    a = jnp.maximum(g_ref[...].reshape(qb * K, h) + p, 0.0)
    m = jnp.dot(a, w2_ref[...], preferred_element_type=jnp.float32)
    agg = jnp.max(m.reshape(qb, K, h), axis=1) + b2_ref[...]
    o = jnp.maximum(
        jnp.dot(agg, wg1_ref[...], preferred_element_type=jnp.float32)
        + bg1_ref[...], 0.0)
    o_ref[...] = jnp.dot(o, wg2_ref[...],
                         preferred_element_type=jnp.float32) + bg2_ref[...]


def _tc_msg(G, RX, RY, RZ, wp, b2, w2, wg1, bg1, wg2, bg2):
    n, _, h = G.shape
    qb = 80
    grid = n // qb
    return pl.pallas_call(
        functools.partial(_msg_kern, qb=qb, h=h),
        grid=(grid,),
        in_specs=[
            pl.BlockSpec((qb, K, h), lambda i: (i, 0, 0)),
            pl.BlockSpec((qb, K), lambda i: (i, 0)),
            pl.BlockSpec((qb, K), lambda i: (i, 0)),
            pl.BlockSpec((qb, K), lambda i: (i, 0)),
            pl.BlockSpec((3, h), lambda i: (0, 0)),
            pl.BlockSpec((1, h), lambda i: (0, 0)),
            pl.BlockSpec((h, h), lambda i: (0, 0)),
            pl.BlockSpec((h, h), lambda i: (0, 0)),
            pl.BlockSpec((1, h), lambda i: (0, 0)),
            pl.BlockSpec((h, h), lambda i: (0, 0)),
            pl.BlockSpec((1, h), lambda i: (0, 0)),
        ],
        out_specs=pl.BlockSpec((qb, h), lambda i: (i, 0)),
        out_shape=jax.ShapeDtypeStruct((n, h), jnp.float32),
    )(G, RX, RY, RZ, wp, b2, w2, wg1, bg1, wg2, bg2)


def kernel(x, pos, batch, feat_w1, feat_b1, feat_w2, feat_b2,
           glob_w1, glob_b1, glob_w2, glob_b2):
    n, f = x.shape
    h = feat_w2.shape[0]
    qpw = ((n + NW - 1) // NW + 7) // 8 * 8
    npad = NW * qpw

    w1x = feat_w1[:f]
    wp = feat_w1[f:]
    g = _tc_linear(x, w1x, feat_b1.reshape(1, h))

    pad = npad - n
    px = jnp.pad(pos[:, 0], (0, pad))
    py = jnp.pad(pos[:, 1], (0, pad))
    pz = jnp.pad(pos[:, 2], (0, pad))
    sqv = jnp.pad(jnp.sum(pos * pos, axis=1), (0, pad))
    def _round(v):
        u = jax.lax.bitcast_convert_type(v, jnp.int32)
        r = (u + 0x7FFF + ((u >> 16) & 1)) & jnp.int32(-65536)
        return jax.lax.bitcast_convert_type(r, jnp.float32)

    pxb = _round(px)
    pyb = _round(py)
    pzb = _round(pz)
    nb = 8
    seg = jnp.searchsorted(batch, jnp.arange(nb + 1, dtype=jnp.int32)
                           ).astype(jnp.int32)
    qlo = jnp.pad(seg[batch], (0, pad))
    qhi = jnp.pad(seg[batch + 1], (0, pad))

    G, RX, RY, RZ = _make_sc_kernel(n, npad, h, qpw)(
        px, py, pz, pxb, pyb, pzb, sqv, qlo, qhi, g)
    RX = RX.reshape(npad, K)
    RY = RY.reshape(npad, K)
    RZ = RZ.reshape(npad, K)

    out = _tc_msg(G, RX, RY, RZ, wp, feat_b2.reshape(1, h), feat_w2,
                  glob_w1, glob_b1.reshape(1, h), glob_w2,
                  glob_b2.reshape(1, h))
    return (out, pos, batch)
